# baseline (speedup 1.0000x reference)
import jax
import jax.numpy as jnp
from jax import lax
from jax.experimental import pallas as pl
from jax.experimental.pallas import tpu as pltpu
from jax.experimental.pallas import tpu_sc as plsc

SEQ = 64
EMBED = 64
NJ = 21
NJP = 32
NT = SEQ * NJP
NC = 2
NS = 16
NW = NC * NS
CHUNK = 128


def _sc_body(p_hbm, c_hbm, pos_hbm, piece_hbm, color_hbm, out_hbm, tbl_hbm,
             pos_v, piece_v, color_v, joint_v, loc_v,
             p_v, c_v, idx_v, gbuf, gsem):
    cid = lax.axis_index("c")
    sid = lax.axis_index("s")
    wid = sid * NC + cid
    rows_per_tile = (4096 * SEQ) // NW
    brows_per_tile = 4096 // NW
    nchunk = rows_per_tile // CHUNK

    pltpu.sync_copy(pos_hbm, pos_v)
    pltpu.sync_copy(piece_hbm, piece_v)
    pltpu.sync_copy(color_hbm, color_v)

    def build_joint(j, _):
        pj = jnp.minimum(j // 3, 6)
        cj = j - (j // 3) * 3
        for k in range(EMBED // 16):
            joint_v[j, pl.ds(k * 16, 16)] = (
                piece_v[pj, pl.ds(k * 16, 16)] + color_v[cj, pl.ds(k * 16, 16)])
        return _
    lax.fori_loop(0, NJP, build_joint, 0)

    def build_row(i, _):
        r = sid * (NT // NS) + i
        s = r // NJP
        j = r - s * NJP
        for k in range(EMBED // 16):
            loc_v[i, pl.ds(k * 16, 16)] = (
                pos_v[s, pl.ds(k * 16, 16)] + joint_v[j, pl.ds(k * 16, 16)])
        return _
    lax.fori_loop(0, NT // NS, build_row, 0)
    pltpu.sync_copy(loc_v, tbl_hbm.at[pl.ds(cid * NT + sid * (NT // NS), NT // NS)])

    base_b = wid * brows_per_tile
    pltpu.sync_copy(p_hbm.at[pl.ds(base_b, brows_per_tile)], p_v)
    pltpu.sync_copy(c_hbm.at[pl.ds(base_b, brows_per_tile)], c_v)
    tbl_base = cid * NT

    def build_idx(i, _):
        r = i // 4
        k = i - r * 4
        pv = p_v[r, pl.ds(k * 16, 16)]
        cv = c_v[r, pl.ds(k * 16, 16)]
        sv = lax.broadcasted_iota(jnp.int32, (16,), 0) + k * 16
        idx_v[i // 8, pl.ds((i - (i // 8) * 8) * 16, 16)] = (
            sv * NJP + pv * 3 + cv + tbl_base)
        return _
    lax.fori_loop(0, rows_per_tile // 16, build_idx, 0)

    plsc.subcore_barrier()

    def gather_cp(j, b):
        return pltpu.make_async_copy(tbl_hbm.at[idx_v.at[j]], gbuf.at[b], gsem)

    gather_cp(0, 0).start()
    out_base = wid * rows_per_tile

    def chunk_step(j, _):
        b = j & 1
        gather_cp(j, b).wait()

        @pl.when(j + 1 < nchunk)
        def _start_next():
            gather_cp(j + 1, 1 - b).start()

        pltpu.sync_copy(gbuf.at[b], out_hbm.at[pl.ds(out_base + j * CHUNK, CHUNK)])
        return _
    lax.fori_loop(0, nchunk, chunk_step, 0)


def kernel(pieces_ids, color_ids, position_emb, piece_emb, color_emb):
    B = pieces_ids.shape[0]
    p32 = pieces_ids.astype(jnp.int32)
    c32 = color_ids.astype(jnp.int32)

    mesh = plsc.VectorSubcoreMesh(core_axis_name="c", subcore_axis_name="s")
    run = pl.kernel(
        _sc_body,
        mesh=mesh,
        compiler_params=pltpu.CompilerParams(use_tc_tiling_on_sc=False),
        out_type=(
            jax.ShapeDtypeStruct((B * SEQ, EMBED), jnp.float32),
            jax.ShapeDtypeStruct((NC * NT, EMBED), jnp.float32),
        ),
        scratch_types=[
            pltpu.VMEM((SEQ, EMBED), jnp.float32),
            pltpu.VMEM((7, EMBED), jnp.float32),
            pltpu.VMEM((3, EMBED), jnp.float32),
            pltpu.VMEM((NJP, EMBED), jnp.float32),
            pltpu.VMEM((NT // NS, EMBED), jnp.float32),
            pltpu.VMEM((B // NW, SEQ), jnp.int32),
            pltpu.VMEM((B // NW, SEQ), jnp.int32),
            pltpu.VMEM((64, CHUNK), jnp.int32),
            pltpu.VMEM((2, CHUNK, EMBED), jnp.float32),
            pltpu.SemaphoreType.DMA,
        ],
    )
    out, _tbl = run(p32, c32, position_emb, piece_emb, color_emb)
    return out.reshape(B, SEQ, EMBED)


# speedup vs baseline: 1.0668x; 1.0668x over previous
import jax
import jax.numpy as jnp
from jax import lax
from jax.experimental import pallas as pl
from jax.experimental.pallas import tpu as pltpu
from jax.experimental.pallas import tpu_sc as plsc

SEQ = 64
EMBED = 64
NJ = 21
NJP = 32
NT = SEQ * NJP
NC = 2
NS = 16
NW = NC * NS
NBUF = 4


def _sc_body(p_hbm, c_hbm, pos_hbm, piece_hbm, color_hbm, out_hbm, tbl_hbm,
             pos_v, piece_v, color_v, joint_v, loc_v,
             p_v, c_v, idx_v, gbuf, gsem):
    cid = lax.axis_index("c")
    sid = lax.axis_index("s")
    wid = sid * NC + cid
    brows_per_tile = 4096 // NW
    nchunk = brows_per_tile // 2

    pltpu.sync_copy(pos_hbm, pos_v)
    pltpu.sync_copy(piece_hbm, piece_v)
    pltpu.sync_copy(color_hbm, color_v)

    def build_joint(j, _):
        pj = jnp.minimum(j // 3, 6)
        cj = j - (j // 3) * 3
        for k in range(EMBED // 16):
            joint_v[j, pl.ds(k * 16, 16)] = (
                piece_v[pj, pl.ds(k * 16, 16)] + color_v[cj, pl.ds(k * 16, 16)])
        return _
    lax.fori_loop(0, NJP, build_joint, 0)

    def build_row(i, _):
        r = sid * (NT // NS) + i
        s = r // NJP
        j = r - s * NJP
        for k in range(EMBED // 16):
            loc_v[i, pl.ds(k * 16, 16)] = (
                pos_v[s, pl.ds(k * 16, 16)] + joint_v[j, pl.ds(k * 16, 16)])
        return _
    lax.fori_loop(0, NT // NS, build_row, 0)
    pltpu.sync_copy(loc_v, tbl_hbm.at[pl.ds(cid * NT + sid * (NT // NS), NT // NS)])

    base_b = wid * brows_per_tile
    pltpu.sync_copy(p_hbm.at[pl.ds(base_b, brows_per_tile)], p_v)
    pltpu.sync_copy(c_hbm.at[pl.ds(base_b, brows_per_tile)], c_v)
    tbl_base = cid * NT

    def build_idx(i, _):
        r = i // 4
        k = i - r * 4
        pv = p_v[r, pl.ds(k * 16, 16)]
        cv = c_v[r, pl.ds(k * 16, 16)]
        sv = lax.broadcasted_iota(jnp.int32, (16,), 0) + k * 16
        idx_v[r // 2, pl.ds((r - (r // 2) * 2) * 64 + k * 16, 16)] = (
            sv * NJP + pv * 3 + cv + tbl_base)
        return _
    lax.fori_loop(0, brows_per_tile * 4, build_idx, 0)

    plsc.subcore_barrier()

    def gather_cp(j, b):
        return pltpu.make_async_copy(
            tbl_hbm.at[idx_v.at[j]], gbuf.at[b], gsem.at[b])

    for b in range(NBUF - 1):
        gather_cp(b, b).start()

    def chunk_step(j, _):
        b = lax.rem(j, NBUF)
        gather_cp(j, b).wait()

        @pl.when(j + NBUF - 1 < nchunk)
        def _start_ahead():
            gather_cp(j + NBUF - 1, lax.rem(j + NBUF - 1, NBUF)).start()

        bb = base_b + j * 2
        pltpu.sync_copy(gbuf.at[b, pl.ds(0, SEQ)], out_hbm.at[bb])
        pltpu.sync_copy(gbuf.at[b, pl.ds(SEQ, SEQ)], out_hbm.at[bb + 1])
        return _
    lax.fori_loop(0, nchunk, chunk_step, 0)


def kernel(pieces_ids, color_ids, position_emb, piece_emb, color_emb):
    B = pieces_ids.shape[0]
    p32 = pieces_ids.astype(jnp.int32)
    c32 = color_ids.astype(jnp.int32)

    mesh = plsc.VectorSubcoreMesh(core_axis_name="c", subcore_axis_name="s")
    run = pl.kernel(
        _sc_body,
        mesh=mesh,
        compiler_params=pltpu.CompilerParams(use_tc_tiling_on_sc=False),
        out_type=(
            jax.ShapeDtypeStruct((B, SEQ, EMBED), jnp.float32),
            jax.ShapeDtypeStruct((NC * NT, EMBED), jnp.float32),
        ),
        scratch_types=[
            pltpu.VMEM((SEQ, EMBED), jnp.float32),
            pltpu.VMEM((7, EMBED), jnp.float32),
            pltpu.VMEM((3, EMBED), jnp.float32),
            pltpu.VMEM((NJP, EMBED), jnp.float32),
            pltpu.VMEM((NT // NS, EMBED), jnp.float32),
            pltpu.VMEM((B // NW, SEQ), jnp.int32),
            pltpu.VMEM((B // NW, SEQ), jnp.int32),
            pltpu.VMEM((B // NW // 2, 2 * SEQ), jnp.int32),
            pltpu.VMEM((NBUF, 2 * SEQ, EMBED), jnp.float32),
            pltpu.SemaphoreType.DMA((NBUF,)),
        ],
    )
    out, _tbl = run(p32, c32, position_emb, piece_emb, color_emb)
    return out
